# Initial kernel scaffold; baseline (speedup 1.0000x reference)
import jax
import jax.numpy as jnp
from jax.experimental import pallas as pl


def kernel(input_ids, attention_mask, token_embedding, codebook):
    raise NotImplementedError("write your pallas kernel here")


# speedup vs baseline: 4.5294x; 4.5294x over previous
import functools

import jax
import jax.numpy as jnp
from jax import lax
from jax.experimental import pallas as pl
from jax.experimental.pallas import tpu as pltpu
from jax.experimental.pallas import tpu_sc as plsc

_B, _S, _D = 16, 2048, 32
_K = 8192
_KMIX = 8
_T = _B * _S
_NC, _NS = 2, 16
_NW = _NC * _NS
_TPW = _T // _NW
_CHUNK = 128

_TT = 1024
_KT = 512


def _sc_mesh():
    return plsc.VectorSubcoreMesh(
        core_axis_name="c", subcore_axis_name="s",
        num_cores=_NC, num_subcores=_NS)


def _embed_gather(ids, emb):
    nchunk = _TPW // _CHUNK

    @functools.partial(
        pl.kernel,
        out_type=jax.ShapeDtypeStruct((_T, _D), jnp.float32),
        mesh=_sc_mesh(),
        compiler_params=pltpu.CompilerParams(use_tc_tiling_on_sc=False),
        scratch_types=[
            pltpu.VMEM((_CHUNK,), jnp.int32),
            pltpu.VMEM((_CHUNK, _D), jnp.float32),
            pltpu.SemaphoreType.DMA,
        ],
    )
    def k(ids_hbm, emb_hbm, out_hbm, idx_v, rows_v, sem):
        wid = lax.axis_index("s") * _NC + lax.axis_index("c")
        t0 = wid * _TPW

        def chunk(c, carry):
            off = t0 + c * _CHUNK
            pltpu.sync_copy(ids_hbm.at[pl.ds(off, _CHUNK)], idx_v)
            pltpu.async_copy(emb_hbm.at[idx_v], rows_v, sem).wait()
            pltpu.sync_copy(rows_v, out_hbm.at[pl.ds(off, _CHUNK)])
            return carry

        lax.fori_loop(0, nchunk, chunk, 0)

    return k(ids, emb)


def _tc_argmin(base2d, maskf, cbt):
    grid = (_T // _TT, _K // _KT)

    def body(base_ref, mask_ref, cbt_ref, out_ref, best_ref, bidx_ref):
        kk = pl.program_id(1)
        base = base_ref[...] * mask_ref[...]
        cb = cbt_ref[...]
        dots = jnp.dot(base, cb, preferred_element_type=jnp.float32)
        c2 = jnp.sum(cb * cb, axis=0, keepdims=True)
        d = c2 - 2.0 * dots
        m = jnp.min(d, axis=1, keepdims=True)
        iota = lax.broadcasted_iota(jnp.int32, (_TT, _KT), 1)
        loc = jnp.min(jnp.where(d == m, iota, jnp.int32(2**30)),
                      axis=1, keepdims=True)
        idx = loc + kk * _KT

        @pl.when(kk == 0)
        def _():
            best_ref[...] = m
            bidx_ref[...] = idx

        @pl.when(kk > 0)
        def _():
            upd = m < best_ref[...]
            bidx_ref[...] = jnp.where(upd, idx, bidx_ref[...])
            best_ref[...] = jnp.where(upd, m, best_ref[...])

        @pl.when(kk == grid[1] - 1)
        def _():
            out_ref[...] = bidx_ref[...]

    return pl.pallas_call(
        body,
        grid=grid,
        in_specs=[
            pl.BlockSpec((_TT, _D), lambda t, k: (t, 0)),
            pl.BlockSpec((_TT, 1), lambda t, k: (t, 0)),
            pl.BlockSpec((_D, _KT), lambda t, k: (0, k)),
        ],
        out_specs=pl.BlockSpec((_TT, 1), lambda t, k: (t, 0)),
        out_shape=jax.ShapeDtypeStruct((_T, 1), jnp.int32),
        scratch_shapes=[
            pltpu.VMEM((_TT, 1), jnp.float32),
            pltpu.VMEM((_TT, 1), jnp.int32),
        ],
    )(base2d, maskf, cbt)


def _mix_gather(qidx, off, codebook):
    tok_per_chunk = _CHUNK // _KMIX
    nchunk = _TPW // tok_per_chunk
    opw = _TPW * _KMIX

    @functools.partial(
        pl.kernel,
        out_type=jax.ShapeDtypeStruct((_T, _D), jnp.float32),
        mesh=_sc_mesh(),
        compiler_params=pltpu.CompilerParams(
            use_tc_tiling_on_sc=False, needs_layout_passes=False),
        scratch_types=[
            pltpu.VMEM((_T,), jnp.int32),
            pltpu.VMEM((opw,), jnp.int32),
            pltpu.VMEM((_CHUNK,), jnp.int32),
            pltpu.VMEM((_CHUNK, _D), jnp.float32),
            pltpu.VMEM((_TPW, _D), jnp.float32),
            pltpu.SemaphoreType.DMA,
        ],
    )
    def k(qidx_hbm, off_hbm, cb_hbm, out_hbm,
          qidx_v, off_v, codes_v, rows_v, out_v, sem):
        wid = lax.axis_index("s") * _NC + lax.axis_index("c")
        pltpu.sync_copy(qidx_hbm, qidx_v)
        pltpu.sync_copy(off_hbm.at[pl.ds(wid * opw, opw)], off_v)

        def chunk(c, carry):
            cbase = c * _CHUNK
            for g in range(_CHUNK // 16):
                off16 = off_v[pl.ds(cbase + g * 16, 16)]
                codes_v[pl.ds(g * 16, 16)] = plsc.load_gather(qidx_v, [off16])
            pltpu.async_copy(cb_hbm.at[codes_v], rows_v, sem).wait()
            for i in range(tok_per_chunk):
                for h in range(_D // 16):
                    acc = rows_v[i * _KMIX, pl.ds(h * 16, 16)]
                    for j in range(1, _KMIX):
                        acc = acc + rows_v[i * _KMIX + j, pl.ds(h * 16, 16)]
                    out_v[c * tok_per_chunk + i, pl.ds(h * 16, 16)] = (
                        acc * (1.0 / _KMIX))
            return carry

        lax.fori_loop(0, nchunk, chunk, 0)
        pltpu.sync_copy(out_v, out_hbm.at[pl.ds(wid * _TPW, _TPW)])

    return k(qidx, off, codebook)


def kernel(input_ids, attention_mask, token_embedding, codebook):
    ids = input_ids.reshape(-1)
    maskf = attention_mask.reshape(-1, 1).astype(jnp.float32)
    base = _embed_gather(ids, token_embedding)
    cbt = codebook.T
    qidx = _tc_argmin(base, maskf, cbt).reshape(-1)
    mix = jax.random.randint(jax.random.key(1), (_B, _S, _KMIX), 0, _B)
    off = (mix * _S
           + jnp.arange(_S, dtype=jnp.int32)[None, :, None]).reshape(-1)
    mixed = _mix_gather(qidx, off.astype(jnp.int32), codebook)
    return mixed.reshape(_B, _S, _D)
